# Initial kernel scaffold; baseline (speedup 1.0000x reference)
import jax
import jax.numpy as jnp
from jax.experimental import pallas as pl


def kernel(x, target):
    raise NotImplementedError("write your pallas kernel here")


# speedup vs baseline: 2.4719x; 2.4719x over previous
import functools
import math

import jax
import jax.numpy as jnp
from jax import lax
from jax.experimental import pallas as pl
from jax.experimental.pallas import tpu as pltpu
from jax.experimental.pallas import tpu_sc as plsc

N_ROWS = 2048
SIZE = 32000
PAD = 0
EPS = 0.1 / (SIZE - 2)
CONF = 0.9
C0 = 0.1 * math.log(EPS) + CONF * math.log(CONF)

NC = 2
NS = 16
NW = NC * NS
RPW = N_ROWS // NW
LANES = 16

BR = 256
BC = 6400
GR = N_ROWS // BR
GC = SIZE // BC


def _sc_body(x_flat_hbm, tgt_hbm, out_hbm, tgt_v, idx_v, g_v, acc_v, sem):
    wid = lax.axis_index("s") * NC + lax.axis_index("c")
    base = wid * RPW
    pltpu.sync_copy(tgt_hbm.at[pl.ds(base, RPW)], tgt_v)
    for j in range(RPW // LANES):
        t = tgt_v[pl.ds(j * LANES, LANES)]
        row = base + (j * LANES + lax.iota(jnp.int32, LANES))
        idx_v[pl.ds(j * LANES, LANES)] = row * SIZE + t
    pltpu.async_copy(x_flat_hbm.at[idx_v], g_v, sem).wait()
    acc = jnp.zeros((LANES,), jnp.float32)
    zero = jnp.zeros((LANES,), jnp.float32)
    for j in range(RPW // LANES):
        t = tgt_v[pl.ds(j * LANES, LANES)]
        g = g_v[pl.ds(j * LANES, LANES)]
        contrib = (EPS - CONF) * g + C0
        acc = acc + jnp.where(t != PAD, contrib, zero)
    acc_v[...] = acc
    pltpu.sync_copy(acc_v, out_hbm.at[wid])


@functools.cache
def _sc_gather():
    return pl.kernel(
        _sc_body,
        mesh=plsc.VectorSubcoreMesh(core_axis_name="c", subcore_axis_name="s"),
        out_type=jax.ShapeDtypeStruct((NW, LANES), jnp.float32),
        scratch_types=[
            pltpu.VMEM((RPW,), jnp.int32),
            pltpu.VMEM((RPW,), jnp.int32),
            pltpu.VMEM((RPW,), jnp.float32),
            pltpu.VMEM((LANES,), jnp.float32),
            pltpu.SemaphoreType.DMA,
        ],
    )


def _tc_body(x_ref, m_ref, sc_ref, out_ref):
    i = pl.program_id(0)
    j = pl.program_id(1)

    @pl.when((i == 0) & (j == 0))
    def _init():
        out_ref[0, 0] = 0.0

    rs = jnp.sum(x_ref[...], axis=1, keepdims=True)
    part = jnp.sum(rs * m_ref[...])
    part = jnp.where(j == 0, part - jnp.sum(x_ref[:, 0:1] * m_ref[...]), part)
    out_ref[0, 0] += part

    @pl.when((i == GR - 1) & (j == GC - 1))
    def _finalize():
        out_ref[0, 0] = jnp.sum(sc_ref[...]) - EPS * out_ref[0, 0]


def kernel(x, target):
    mask = (target != PAD).astype(jnp.float32).reshape(N_ROWS, 1)
    sc_part = _sc_gather()(x.reshape(-1), target)
    out = pl.pallas_call(
        _tc_body,
        grid=(GR, GC),
        in_specs=[
            pl.BlockSpec((BR, BC), lambda i, j: (i, j)),
            pl.BlockSpec((BR, 1), lambda i, j: (i, 0)),
            pl.BlockSpec((NW, LANES), lambda i, j: (0, 0)),
        ],
        out_specs=pl.BlockSpec(memory_space=pltpu.SMEM),
        out_shape=jax.ShapeDtypeStruct((1, 1), jnp.float32),
        compiler_params=pltpu.CompilerParams(
            dimension_semantics=("arbitrary", "arbitrary"),
        ),
    )(x, mask, sc_part)
    return out.reshape(())
